# baseline (speedup 1.0000x reference)
import jax
import jax.numpy as jnp
from jax.experimental import pallas as pl
from jax.experimental.pallas import tpu as pltpu

D_MODEL = 512
S = 64
SS = S * S
D2 = D_MODEL // 2


def _build_pos_kernel(row_ref, col_ref, pos_ref):
    colT = col_ref[...].T
    rowT = row_ref[...].T
    pos_col = jnp.broadcast_to(colT[:, None, :], (D2, S, S))
    pos_row = jnp.broadcast_to(rowT[:, :, None], (D2, S, S))
    pos_ref[...] = jnp.concatenate([pos_col, pos_row], axis=0)


def _add_kernel(x_ref, pos_ref, out_ref):
    out_ref[0] = x_ref[0] + pos_ref[...]


def kernel(x, row_embed, col_embed):
    B = x.shape[0]
    pos = pl.pallas_call(
        _build_pos_kernel,
        out_shape=jax.ShapeDtypeStruct((D_MODEL, S, S), x.dtype),
    )(row_embed, col_embed)

    return pl.pallas_call(
        _add_kernel,
        grid=(4, B),
        in_specs=[
            pl.BlockSpec((1, D_MODEL // 4, S, S), lambda ci, b: (b, ci, 0, 0)),
            pl.BlockSpec((D_MODEL // 4, S, S), lambda ci, b: (ci, 0, 0)),
        ],
        out_specs=pl.BlockSpec((1, D_MODEL // 4, S, S), lambda ci, b: (b, ci, 0, 0)),
        out_shape=jax.ShapeDtypeStruct(x.shape, x.dtype),
        compiler_params=pltpu.CompilerParams(
            dimension_semantics=("parallel", "parallel"),
        ),
    )(x, pos)


# speedup vs baseline: 6.7224x; 6.7224x over previous
import jax
import jax.numpy as jnp
from jax.experimental import pallas as pl
from jax.experimental.pallas import tpu as pltpu

D_MODEL = 512
S = 64
D2 = D_MODEL // 2


def _build_pos_kernel(row_ref, col_ref, pos_ref):
    pos_col = jnp.broadcast_to(col_ref[...][None, :, :], (S, S, D2))
    pos_row = jnp.broadcast_to(row_ref[...][:, None, :], (S, S, D2))
    pos_ref[...] = jnp.concatenate([pos_col, pos_row], axis=-1)


def _add_kernel(x_ref, pos_ref, out_ref):
    out_ref[0] = x_ref[0] + pos_ref[...]


def kernel(x, row_embed, col_embed):
    B = x.shape[0]
    pos = pl.pallas_call(
        _build_pos_kernel,
        out_shape=jax.ShapeDtypeStruct((S, S, D_MODEL), x.dtype),
    )(row_embed, col_embed)

    xt = jnp.transpose(x, (0, 2, 3, 1))
    out_t = pl.pallas_call(
        _add_kernel,
        grid=(B,),
        in_specs=[
            pl.BlockSpec((1, S, S, D_MODEL), lambda b: (b, 0, 0, 0)),
            pl.BlockSpec((S, S, D_MODEL), lambda b: (0, 0, 0)),
        ],
        out_specs=pl.BlockSpec((1, S, S, D_MODEL), lambda b: (b, 0, 0, 0)),
        out_shape=jax.ShapeDtypeStruct((B, S, S, D_MODEL), x.dtype),
        compiler_params=pltpu.CompilerParams(
            dimension_semantics=("parallel",),
        ),
    )(xt, pos)
    return jnp.transpose(out_t, (0, 3, 1, 2))


# speedup vs baseline: 7.0187x; 1.0441x over previous
import jax
import jax.numpy as jnp
from jax.experimental import pallas as pl
from jax.experimental.pallas import tpu as pltpu

D_MODEL = 512
S = 64
D2 = D_MODEL // 2


def _add_pos_kernel(x_ref, row_ref, col_ref, out_ref):
    pos_col = jnp.broadcast_to(col_ref[...][None, :, :], (S, S, D2))
    pos_row = jnp.broadcast_to(row_ref[...][:, None, :], (S, S, D2))
    out_ref[0, :, :, :D2] = x_ref[0, :, :, :D2] + pos_col
    out_ref[0, :, :, D2:] = x_ref[0, :, :, D2:] + pos_row


def kernel(x, row_embed, col_embed):
    B = x.shape[0]
    xt = jnp.transpose(x, (0, 2, 3, 1))
    out_t = pl.pallas_call(
        _add_pos_kernel,
        grid=(B,),
        in_specs=[
            pl.BlockSpec((1, S, S, D_MODEL), lambda b: (b, 0, 0, 0)),
            pl.BlockSpec((S, D2), lambda b: (0, 0)),
            pl.BlockSpec((S, D2), lambda b: (0, 0)),
        ],
        out_specs=pl.BlockSpec((1, S, S, D_MODEL), lambda b: (b, 0, 0, 0)),
        out_shape=jax.ShapeDtypeStruct((B, S, S, D_MODEL), x.dtype),
        compiler_params=pltpu.CompilerParams(
            dimension_semantics=("parallel",),
        ),
    )(xt, row_embed, col_embed)
    return jnp.transpose(out_t, (0, 3, 1, 2))
